# Initial kernel scaffold; baseline (speedup 1.0000x reference)
import jax
import jax.numpy as jnp
from jax.experimental import pallas as pl


def kernel(D_output_fake, fake_data, real_data):
    raise NotImplementedError("write your pallas kernel here")


# speedup vs baseline: 261.8029x; 261.8029x over previous
import jax
import jax.numpy as jnp
from jax.experimental import pallas as pl

_ADV_W = 0.6
_NORM_W = 0.05
_DATA_W = 0.25


def _loss_kernel(d_ref, fake_ref, real_ref, out_ref):
    diff = real_ref[...] - fake_ref[...]
    sq = diff * diff
    ch = jax.lax.broadcasted_iota(jnp.int32, sq.shape, 2)
    w = jnp.where(ch < 3, _DATA_W, _NORM_W)
    wsum = jnp.sum(sq * w)
    n_per_slice = sq.shape[0] * sq.shape[1] * 3
    adv = -jnp.sum(jnp.log(d_ref[...] + 1e-08)) / d_ref.size
    total = _ADV_W * adv + wsum / n_per_slice
    out_ref[...] = jnp.reshape(total, (1, 1))


def kernel(D_output_fake, fake_data, real_data):
    out = pl.pallas_call(
        _loss_kernel,
        out_shape=jax.ShapeDtypeStruct((1, 1), jnp.float32),
    )(D_output_fake, fake_data, real_data)
    return out[0, 0]
